# baseline (speedup 1.0000x reference)
import jax
import jax.numpy as jnp
from jax.experimental import pallas as pl
from jax.experimental.pallas import tpu as pltpu


_M = 0.999
_T_MOCO = 0.07
_W = 8192


def _bdot(a, b):
    return jnp.dot(a.astype(jnp.bfloat16), b.astype(jnp.bfloat16),
                   preferred_element_type=jnp.float32)


def _body(imq_ref, imk_ref, wf_ref, bf_ref, wc_ref, bc_ref, mwf_ref, mbf_ref,
          f_ref, feats_ref, logq_ref, ins_ref, k_ref, fp_scr):
    inv_t = 1.0 / _T_MOCO
    c, kp = fp_scr.shape
    i = pl.program_id(0)
    j = pl.program_id(1)

    @pl.when(jnp.logical_and(i == 0, j == 0))
    def _():
        fp_scr[:, 0:1] = jnp.zeros((c, 1), jnp.bfloat16)
        fp_scr[:, 1:_W + 1] = f_ref[...].astype(jnp.bfloat16)

    @pl.when(jnp.logical_and(i == 0, j == 1))
    def _():
        fp_scr[:, _W + 1:kp] = f_ref[...].astype(jnp.bfloat16)

    feats = _bdot(imq_ref[...], wf_ref[...]) + bf_ref[...]
    feats_ref[...] = feats
    logq_ref[...] = _bdot(feats, wc_ref[...]) + bc_ref[...]
    qn = jnp.sqrt(jnp.sum(feats * feats, axis=1, keepdims=True))
    q = feats / jnp.maximum(qn, 1e-12)

    mw_new = mwf_ref[...] * _M + wf_ref[...] * (1.0 - _M)
    mb_new = mbf_ref[...] * _M + bf_ref[...] * (1.0 - _M)
    kf = _bdot(imk_ref[...], mw_new) + mb_new
    kn = jnp.sqrt(jnp.sum(kf * kf, axis=1, keepdims=True))
    kv = kf / jnp.maximum(kn, 1e-12)
    k_ref[...] = kv

    qs = (q * inv_t).astype(jnp.bfloat16)

    @pl.when(j == 0)
    def _():
        ins_ref[...] = jnp.dot(qs, fp_scr[:, 0:_W],
                               preferred_element_type=jnp.float32)
        ins_ref[:, 0:1] = jnp.sum(q * kv, axis=1, keepdims=True) * inv_t

    @pl.when(j == 1)
    def _():
        ins_ref[...] = jnp.dot(qs, fp_scr[:, _W:2 * _W],
                               preferred_element_type=jnp.float32)

    @pl.when(j == 2)
    def _():
        ins_ref[:, 0:1] = jnp.dot(qs, fp_scr[:, 2 * _W:2 * _W + 1],
                                  preferred_element_type=jnp.float32)


@jax.jit
def kernel(im_q, im_k, W_F, b_F, W_C, b_C, mW_F, mb_F, mW_C, mb_C,
           features_mem):
    B, D = im_q.shape
    C = W_F.shape[1]
    NC = W_C.shape[1]
    K = features_mem.shape[1]
    KP = K + 1

    bf2 = b_F.reshape(1, C)
    bc2 = b_C.reshape(1, NC)
    mbf2 = mb_F.reshape(1, C)

    BR = 256
    NJ = 3

    def f_idx(i, j):
        return (0, jnp.where(i == 0, jnp.minimum(j, 1), 1))

    feats_q, logits_q, logits_ins, k = pl.pallas_call(
        _body,
        grid=(B // BR, NJ),
        in_specs=[
            pl.BlockSpec((BR, D), lambda i, j: (i, 0)),
            pl.BlockSpec((BR, D), lambda i, j: (i, 0)),
            pl.BlockSpec((D, C), lambda i, j: (0, 0)),
            pl.BlockSpec((1, C), lambda i, j: (0, 0)),
            pl.BlockSpec((C, NC), lambda i, j: (0, 0)),
            pl.BlockSpec((1, NC), lambda i, j: (0, 0)),
            pl.BlockSpec((D, C), lambda i, j: (0, 0)),
            pl.BlockSpec((1, C), lambda i, j: (0, 0)),
            pl.BlockSpec((C, _W), f_idx),
        ],
        scratch_shapes=[pltpu.VMEM((C, KP), jnp.bfloat16)],
        out_specs=[
            pl.BlockSpec((BR, C), lambda i, j: (i, 0)),
            pl.BlockSpec((BR, NC), lambda i, j: (i, 0)),
            pl.BlockSpec((BR, _W), lambda i, j: (i, j)),
            pl.BlockSpec((BR, C), lambda i, j: (i, 0)),
        ],
        out_shape=[
            jax.ShapeDtypeStruct((B, C), jnp.float32),
            jax.ShapeDtypeStruct((B, NC), jnp.float32),
            jax.ShapeDtypeStruct((B, KP), jnp.float32),
            jax.ShapeDtypeStruct((B, C), jnp.float32),
        ],
        compiler_params=pltpu.CompilerParams(
            dimension_semantics=("arbitrary", "arbitrary")),
    )(im_q, im_k, W_F, bf2, W_C, bc2, mW_F, mbf2, features_mem)

    return (feats_q, logits_q, logits_ins, k)


# speedup vs baseline: 1.0040x; 1.0040x over previous
import jax
import jax.numpy as jnp
from jax.experimental import pallas as pl
from jax.experimental.pallas import tpu as pltpu


_M = 0.999
_T_MOCO = 0.07
_W = 8192


def _bdot(a, b):
    return jnp.dot(a.astype(jnp.bfloat16), b.astype(jnp.bfloat16),
                   preferred_element_type=jnp.float32)


def _body(imq_ref, imk_ref, wf_ref, bf_ref, wc_ref, bc_ref, mwf_ref, mbf_ref,
          f_ref, feats_ref, logq_ref, ins_ref, k_ref, fp_scr, qs_scr,
          lpos_scr):
    inv_t = 1.0 / _T_MOCO
    c, kp = fp_scr.shape
    i = pl.program_id(0)
    j = pl.program_id(1)

    @pl.when(jnp.logical_and(i == 0, j == 0))
    def _():
        fp_scr[:, 0:1] = jnp.zeros((c, 1), jnp.bfloat16)
        fp_scr[:, 1:_W + 1] = f_ref[...].astype(jnp.bfloat16)

    @pl.when(jnp.logical_and(i == 0, j == 1))
    def _():
        fp_scr[:, _W + 1:kp] = f_ref[...].astype(jnp.bfloat16)

    @pl.when(j == 0)
    def _():
        feats = _bdot(imq_ref[...], wf_ref[...]) + bf_ref[...]
        feats_ref[...] = feats
        logq_ref[...] = _bdot(feats, wc_ref[...]) + bc_ref[...]
        qn = jnp.sqrt(jnp.sum(feats * feats, axis=1, keepdims=True))
        q = feats / jnp.maximum(qn, 1e-12)

        mw_new = mwf_ref[...] * _M + wf_ref[...] * (1.0 - _M)
        mb_new = mbf_ref[...] * _M + bf_ref[...] * (1.0 - _M)
        kf = _bdot(imk_ref[...], mw_new) + mb_new
        kn = jnp.sqrt(jnp.sum(kf * kf, axis=1, keepdims=True))
        kv = kf / jnp.maximum(kn, 1e-12)
        k_ref[...] = kv

        qs_scr[...] = (q * inv_t).astype(jnp.bfloat16)
        lpos_scr[...] = jnp.sum(q * kv, axis=1, keepdims=True) * inv_t

    qs = qs_scr[...]

    @pl.when(j == 0)
    def _():
        ins_ref[...] = jnp.dot(qs, fp_scr[:, 0:_W],
                               preferred_element_type=jnp.float32)
        ins_ref[:, 0:1] = lpos_scr[...]

    @pl.when(j == 1)
    def _():
        ins_ref[...] = jnp.dot(qs, fp_scr[:, _W:2 * _W],
                               preferred_element_type=jnp.float32)

    @pl.when(j == 2)
    def _():
        ins_ref[:, 0:1] = jnp.dot(qs, fp_scr[:, 2 * _W:2 * _W + 1],
                                  preferred_element_type=jnp.float32)


@jax.jit
def kernel(im_q, im_k, W_F, b_F, W_C, b_C, mW_F, mb_F, mW_C, mb_C,
           features_mem):
    B, D = im_q.shape
    C = W_F.shape[1]
    NC = W_C.shape[1]
    K = features_mem.shape[1]
    KP = K + 1

    bf2 = b_F.reshape(1, C)
    bc2 = b_C.reshape(1, NC)
    mbf2 = mb_F.reshape(1, C)

    BR = 256
    NJ = 3

    def f_idx(i, j):
        return (0, jnp.where(i == 0, jnp.minimum(j, 1), 1))

    feats_q, logits_q, logits_ins, k = pl.pallas_call(
        _body,
        grid=(B // BR, NJ),
        in_specs=[
            pl.BlockSpec((BR, D), lambda i, j: (i, 0)),
            pl.BlockSpec((BR, D), lambda i, j: (i, 0)),
            pl.BlockSpec((D, C), lambda i, j: (0, 0)),
            pl.BlockSpec((1, C), lambda i, j: (0, 0)),
            pl.BlockSpec((C, NC), lambda i, j: (0, 0)),
            pl.BlockSpec((1, NC), lambda i, j: (0, 0)),
            pl.BlockSpec((D, C), lambda i, j: (0, 0)),
            pl.BlockSpec((1, C), lambda i, j: (0, 0)),
            pl.BlockSpec((C, _W), f_idx),
        ],
        scratch_shapes=[
            pltpu.VMEM((C, KP), jnp.bfloat16),
            pltpu.VMEM((BR, C), jnp.bfloat16),
            pltpu.VMEM((BR, 1), jnp.float32),
        ],
        out_specs=[
            pl.BlockSpec((BR, C), lambda i, j: (i, 0)),
            pl.BlockSpec((BR, NC), lambda i, j: (i, 0)),
            pl.BlockSpec((BR, _W), lambda i, j: (i, j)),
            pl.BlockSpec((BR, C), lambda i, j: (i, 0)),
        ],
        out_shape=[
            jax.ShapeDtypeStruct((B, C), jnp.float32),
            jax.ShapeDtypeStruct((B, NC), jnp.float32),
            jax.ShapeDtypeStruct((B, KP), jnp.float32),
            jax.ShapeDtypeStruct((B, C), jnp.float32),
        ],
        compiler_params=pltpu.CompilerParams(
            dimension_semantics=("arbitrary", "arbitrary")),
    )(im_q, im_k, W_F, bf2, W_C, bc2, mW_F, mbf2, features_mem)

    return (feats_q, logits_q, logits_ins, k)


# speedup vs baseline: 1.1212x; 1.1167x over previous
import jax
import jax.numpy as jnp
from jax.experimental import pallas as pl
from jax.experimental.pallas import tpu as pltpu


_M = 0.999
_T_MOCO = 0.07


def _bdot(a, b):
    return jnp.dot(a.astype(jnp.bfloat16), b.astype(jnp.bfloat16),
                   preferred_element_type=jnp.float32)


def _body(imq_ref, imk_ref, wf_ref, bf_ref, wc_ref, bc_ref, mwf_ref, mbf_ref,
          f_ref, feats_ref, logq_ref, ins_ref, k_ref, fp_scr):
    inv_t = 1.0 / _T_MOCO
    c, kp = fp_scr.shape

    @pl.when(pl.program_id(0) == 0)
    def _():
        fp_scr[:, 0:1] = jnp.zeros((c, 1), jnp.bfloat16)
        fp_scr[:, 1:kp] = f_ref[...].astype(jnp.bfloat16)

    feats = _bdot(imq_ref[...], wf_ref[...]) + bf_ref[...]
    feats_ref[...] = feats
    logq_ref[...] = _bdot(feats, wc_ref[...]) + bc_ref[...]
    qn = jnp.sqrt(jnp.sum(feats * feats, axis=1, keepdims=True))
    q = feats / jnp.maximum(qn, 1e-12)

    mw_new = mwf_ref[...] * _M + wf_ref[...] * (1.0 - _M)
    mb_new = mbf_ref[...] * _M + bf_ref[...] * (1.0 - _M)
    kf = _bdot(imk_ref[...], mw_new) + mb_new
    kn = jnp.sqrt(jnp.sum(kf * kf, axis=1, keepdims=True))
    kv = kf / jnp.maximum(kn, 1e-12)
    k_ref[...] = kv

    qs = (q * inv_t).astype(jnp.bfloat16)
    ins_ref[...] = jnp.dot(qs, fp_scr[...],
                           preferred_element_type=jnp.float32)
    ins_ref[:, 0:1] = jnp.sum(q * kv, axis=1, keepdims=True) * inv_t


@jax.jit
def kernel(im_q, im_k, W_F, b_F, W_C, b_C, mW_F, mb_F, mW_C, mb_C,
           features_mem):
    B, D = im_q.shape
    C = W_F.shape[1]
    NC = W_C.shape[1]
    K = features_mem.shape[1]
    KP = K + 1

    bf2 = b_F.reshape(1, C)
    bc2 = b_C.reshape(1, NC)
    mbf2 = mb_F.reshape(1, C)

    BR = 256
    feats_q, logits_q, logits_ins, k = pl.pallas_call(
        _body,
        grid=(B // BR,),
        in_specs=[
            pl.BlockSpec((BR, D), lambda i: (i, 0)),
            pl.BlockSpec((BR, D), lambda i: (i, 0)),
            pl.BlockSpec((D, C), lambda i: (0, 0)),
            pl.BlockSpec((1, C), lambda i: (0, 0)),
            pl.BlockSpec((C, NC), lambda i: (0, 0)),
            pl.BlockSpec((1, NC), lambda i: (0, 0)),
            pl.BlockSpec((D, C), lambda i: (0, 0)),
            pl.BlockSpec((1, C), lambda i: (0, 0)),
            pl.BlockSpec((C, K), lambda i: (0, 0)),
        ],
        scratch_shapes=[pltpu.VMEM((C, KP), jnp.bfloat16)],
        out_specs=[
            pl.BlockSpec((BR, C), lambda i: (i, 0)),
            pl.BlockSpec((BR, NC), lambda i: (i, 0)),
            pl.BlockSpec((BR, KP), lambda i: (i, 0)),
            pl.BlockSpec((BR, C), lambda i: (i, 0)),
        ],
        out_shape=[
            jax.ShapeDtypeStruct((B, C), jnp.float32),
            jax.ShapeDtypeStruct((B, NC), jnp.float32),
            jax.ShapeDtypeStruct((B, KP), jnp.float32),
            jax.ShapeDtypeStruct((B, C), jnp.float32),
        ],
        compiler_params=pltpu.CompilerParams(
            dimension_semantics=("arbitrary",)),
    )(im_q, im_k, W_F, bf2, W_C, bc2, mW_F, mbf2, features_mem)

    return (feats_q, logits_q, logits_ins, k)
